# Initial kernel scaffold; baseline (speedup 1.0000x reference)
import jax
import jax.numpy as jnp
from jax.experimental import pallas as pl


def kernel(h, us, vs, ws, a, b):
    raise NotImplementedError("write your pallas kernel here")


# speedup vs baseline: 1.0928x; 1.0928x over previous
import jax
import jax.numpy as jnp
from jax import lax
from jax.experimental import pallas as pl
from jax.experimental.pallas import tpu as pltpu
from jax.experimental.pallas import tpu_sc as plsc

_N_NODES = 10000
_N_EDGES = 320000
_D = 128
_L = 16
_NC = 2
_NS = 16
_NW = _NC * _NS
_K = 128
_N_CHUNKS = _N_EDGES // _K
_CHUNKS_PER_W = -(-_N_CHUNKS // _NW)
_UNROLL = 8


def _body(ah_hbm, bh_hbm, us_hbm, vs_hbm, ws_hbm, out_hbm,
          idx_u, idx_v, ws_v, rows_u, rows_v, out_v, sem):
    wid = lax.axis_index("s") * _NC + lax.axis_index("c")
    lane = lax.iota(jnp.int32, _L)

    def chunk_body(i, carry):
        c = wid + i * _NW

        @pl.when(c < _N_CHUNKS)
        def _():
            base = c * _K
            pltpu.sync_copy(us_hbm.at[pl.ds(base, _K)], idx_u)
            pltpu.sync_copy(vs_hbm.at[pl.ds(base, _K)], idx_v)
            pltpu.sync_copy(ws_hbm.at[pl.ds(base, _K)], ws_v)
            cp_u = pltpu.async_copy(ah_hbm.at[idx_u], rows_u, sem)
            cp_v = pltpu.async_copy(bh_hbm.at[idx_v], rows_v, sem)
            cp_u.wait()
            cp_v.wait()

            for g in range(_K // _L):
                row = lane + (g * _L)

                def dim_body(d0, acc):
                    for k in range(_UNROLL):
                        col = jnp.full((_L,), d0 * _UNROLL + k, jnp.int32)
                        gu = plsc.load_gather(rows_u, [row, col])
                        gv = plsc.load_gather(rows_v, [row, col])
                        cmb = gu + gv
                        acc = acc + cmb * cmb
                    return acc

                acc = lax.fori_loop(0, _D // _UNROLL, dim_body,
                                    jnp.zeros((_L,), jnp.float32))
                w = ws_v[pl.ds(g * _L, _L)]
                out_v[pl.ds(g * _L, _L)] = w / (1.0 + jnp.exp(10.0 - acc))

            pltpu.sync_copy(out_v, out_hbm.at[pl.ds(base, _K)])

        return carry

    lax.fori_loop(0, _CHUNKS_PER_W, chunk_body, 0)


@jax.jit
def _scorer(ah, bh, us, vs, ws):
    mesh = plsc.VectorSubcoreMesh(
        core_axis_name="c", subcore_axis_name="s",
        num_cores=_NC, num_subcores=_NS)
    return pl.kernel(
        _body,
        out_type=jax.ShapeDtypeStruct((_N_EDGES,), jnp.float32),
        mesh=mesh,
        compiler_params=pltpu.CompilerParams(needs_layout_passes=False),
        scratch_types=[
            pltpu.VMEM((_K,), jnp.int32),
            pltpu.VMEM((_K,), jnp.int32),
            pltpu.VMEM((_K,), jnp.float32),
            pltpu.VMEM((_K, _D), jnp.float32),
            pltpu.VMEM((_K, _D), jnp.float32),
            pltpu.VMEM((_K,), jnp.float32),
            pltpu.SemaphoreType.DMA,
        ],
    )(ah, bh, us, vs, ws)


def kernel(h, us, vs, ws, a, b):
    return _scorer(h * a, h * b, us, vs, ws)


# speedup vs baseline: 4.1498x; 3.7976x over previous
import jax
import jax.numpy as jnp
from jax import lax
from jax.experimental import pallas as pl
from jax.experimental.pallas import tpu as pltpu
from jax.experimental.pallas import tpu_sc as plsc

_N_NODES = 10000
_N_EDGES = 320000
_D = 128
_L = 16
_NC = 2
_NS = 16
_NW = _NC * _NS
_K = 128
_N_CHUNKS = _N_EDGES // _K
_CHUNKS_PER_W = -(-_N_CHUNKS // _NW)
_UNROLL = 8


def _body(ah_hbm, bh_hbm, us_hbm, vs_hbm, ws_hbm, out_hbm,
          idx_u, idx_v, ws_v, rows_u, rows_v, out_v, sem):
    wid = lax.axis_index("s") * _NC + lax.axis_index("c")
    lane = lax.iota(jnp.int32, _L)

    def chunk_body(i, carry):
        c = wid + i * _NW

        @pl.when(c < _N_CHUNKS)
        def _():
            base = c * _K
            pltpu.sync_copy(us_hbm.at[pl.ds(base, _K)], idx_u)
            pltpu.sync_copy(vs_hbm.at[pl.ds(base, _K)], idx_v)
            pltpu.sync_copy(ws_hbm.at[pl.ds(base, _K)], ws_v)
            cp_u = pltpu.async_copy(ah_hbm.at[idx_u], rows_u, sem)
            cp_v = pltpu.async_copy(bh_hbm.at[idx_v], rows_v, sem)
            cp_u.wait()
            cp_v.wait()

            for g in range(_K // _L):
                row = lane + (g * _L)

                def dim_body(d0, acc):
                    for k in range(_UNROLL):
                        col = (lane + (d0 * _UNROLL + k)) & (_D - 1)
                        gu = plsc.load_gather(rows_u, [row, col])
                        gv = plsc.load_gather(rows_v, [row, col])
                        cmb = gu + gv
                        acc = acc + cmb * cmb
                    return acc

                acc = lax.fori_loop(0, _D // _UNROLL, dim_body,
                                    jnp.zeros((_L,), jnp.float32))
                w = ws_v[pl.ds(g * _L, _L)]
                out_v[pl.ds(g * _L, _L)] = w / (1.0 + jnp.exp(10.0 - acc))

            pltpu.sync_copy(out_v, out_hbm.at[pl.ds(base, _K)])

        return carry

    lax.fori_loop(0, _CHUNKS_PER_W, chunk_body, 0)


@jax.jit
def _scorer(ah, bh, us, vs, ws):
    mesh = plsc.VectorSubcoreMesh(
        core_axis_name="c", subcore_axis_name="s",
        num_cores=_NC, num_subcores=_NS)
    return pl.kernel(
        _body,
        out_type=jax.ShapeDtypeStruct((_N_EDGES,), jnp.float32),
        mesh=mesh,
        compiler_params=pltpu.CompilerParams(needs_layout_passes=False),
        scratch_types=[
            pltpu.VMEM((_K,), jnp.int32),
            pltpu.VMEM((_K,), jnp.int32),
            pltpu.VMEM((_K,), jnp.float32),
            pltpu.VMEM((_K, _D), jnp.float32),
            pltpu.VMEM((_K, _D), jnp.float32),
            pltpu.VMEM((_K,), jnp.float32),
            pltpu.SemaphoreType.DMA,
        ],
    )(ah, bh, us, vs, ws)


def kernel(h, us, vs, ws, a, b):
    return _scorer(h * a, h * b, us, vs, ws)


# speedup vs baseline: 8.8572x; 2.1343x over previous
import jax
import jax.numpy as jnp
from jax import lax
from jax.experimental import pallas as pl
from jax.experimental.pallas import tpu as pltpu
from jax.experimental.pallas import tpu_sc as plsc

_N_NODES = 10000
_N_EDGES = 320000
_D = 128
_L = 16
_NC = 2
_NS = 16
_NW = _NC * _NS
_K = 128
_N_CHUNKS = _N_EDGES // _K
_CHUNKS_PER_W = -(-_N_CHUNKS // _NW)
_UNROLL = 8


def _body(ah_hbm, bh_hbm, us_hbm, vs_hbm, ws_hbm, out_hbm,
          idx_u0, idx_u1, idx_v0, idx_v1, ws0, ws1,
          rows_u0, rows_u1, rows_v0, rows_v1, out0, out1,
          sem_idx, sem_rows, sem_out):
    wid = lax.axis_index("s") * _NC + lax.axis_index("c")
    lane = lax.iota(jnp.int32, _L)
    idx_u = (idx_u0, idx_u1)
    idx_v = (idx_v0, idx_v1)
    ws_b = (ws0, ws1)
    rows_u = (rows_u0, rows_u1)
    rows_v = (rows_v0, rows_v1)
    out_b = (out0, out1)

    def chunk_of(m):
        return wid + m * _NW

    def idx_copies(m, p):
        base = chunk_of(m) * _K
        return (
            pltpu.make_async_copy(us_hbm.at[pl.ds(base, _K)], idx_u[p], sem_idx),
            pltpu.make_async_copy(vs_hbm.at[pl.ds(base, _K)], idx_v[p], sem_idx),
            pltpu.make_async_copy(ws_hbm.at[pl.ds(base, _K)], ws_b[p], sem_idx),
        )

    def row_copies(p):
        return (
            pltpu.make_async_copy(ah_hbm.at[idx_u[p]], rows_u[p], sem_rows),
            pltpu.make_async_copy(bh_hbm.at[idx_v[p]], rows_v[p], sem_rows),
        )

    def out_copy(m, p):
        base = chunk_of(m) * _K
        return pltpu.make_async_copy(
            out_b[p], out_hbm.at[pl.ds(base, _K)], sem_out)

    def compute(p):
        for g in range(_K // _L):
            row = lane + (g * _L)

            def dim_body(d0, acc):
                for k in range(_UNROLL):
                    col = (lane + (d0 * _UNROLL + k)) & (_D - 1)
                    gu = plsc.load_gather(rows_u[p], [row, col])
                    gv = plsc.load_gather(rows_v[p], [row, col])
                    cmb = gu + gv
                    acc = acc + cmb * cmb
                return acc

            acc = lax.fori_loop(0, _D // _UNROLL, dim_body,
                                jnp.zeros((_L,), jnp.float32))
            w = ws_b[p][pl.ds(g * _L, _L)]
            out_b[p][pl.ds(g * _L, _L)] = w / (1.0 + jnp.exp(10.0 - acc))

    for cp in idx_copies(0, 0):
        cp.start()
    for cp in idx_copies(0, 0):
        cp.wait()
    for cp in row_copies(0):
        cp.start()
    for cp in idx_copies(1, 1):
        cp.start()

    def process(m, p):
        live = chunk_of(m) < _N_CHUNKS

        @pl.when(jnp.logical_and(live, chunk_of(m + 1) < _N_CHUNKS))
        def _():
            for cp in idx_copies(m + 1, 1 - p):
                cp.wait()
            for cp in row_copies(1 - p):
                cp.start()

        @pl.when(live)
        def _():
            for cp in row_copies(p):
                cp.wait()

        @pl.when(jnp.logical_and(live, m >= 2))
        def _():
            out_copy(m - 2, p).wait()

        @pl.when(live)
        def _():
            compute(p)
            out_copy(m, p).start()

        @pl.when(chunk_of(m + 2) < _N_CHUNKS)
        def _():
            for cp in idx_copies(m + 2, p):
                cp.start()

    def pair_body(i2, carry):
        process(i2 * 2, 0)
        process(i2 * 2 + 1, 1)
        return carry

    lax.fori_loop(0, (_CHUNKS_PER_W + 1) // 2, pair_body, 0)

    out_copy(0, 0).wait()
    out_copy(1, 1).wait()


@jax.jit
def _scorer(ah, bh, us, vs, ws):
    mesh = plsc.VectorSubcoreMesh(
        core_axis_name="c", subcore_axis_name="s",
        num_cores=_NC, num_subcores=_NS)
    return pl.kernel(
        _body,
        out_type=jax.ShapeDtypeStruct((_N_EDGES,), jnp.float32),
        mesh=mesh,
        compiler_params=pltpu.CompilerParams(needs_layout_passes=False),
        scratch_types=[
            pltpu.VMEM((_K,), jnp.int32),
            pltpu.VMEM((_K,), jnp.int32),
            pltpu.VMEM((_K,), jnp.int32),
            pltpu.VMEM((_K,), jnp.int32),
            pltpu.VMEM((_K,), jnp.float32),
            pltpu.VMEM((_K,), jnp.float32),
            pltpu.VMEM((_K, _D), jnp.float32),
            pltpu.VMEM((_K, _D), jnp.float32),
            pltpu.VMEM((_K, _D), jnp.float32),
            pltpu.VMEM((_K, _D), jnp.float32),
            pltpu.VMEM((_K,), jnp.float32),
            pltpu.VMEM((_K,), jnp.float32),
            pltpu.SemaphoreType.DMA,
            pltpu.SemaphoreType.DMA,
            pltpu.SemaphoreType.DMA,
        ],
    )(ah, bh, us, vs, ws)


def kernel(h, us, vs, ws, a, b):
    return _scorer(h * a, h * b, us, vs, ws)


# speedup vs baseline: 10.0658x; 1.1365x over previous
import jax
import jax.numpy as jnp
from jax import lax
from jax.experimental import pallas as pl
from jax.experimental.pallas import tpu as pltpu
from jax.experimental.pallas import tpu_sc as plsc

_N_NODES = 10000
_N_EDGES = 320000
_D = 128
_L = 16
_NC = 2
_NS = 16
_NW = _NC * _NS
_K = 128
_N_CHUNKS = _N_EDGES // _K
_BASE_CHUNKS = _N_CHUNKS // _NW
_EXTRA_W = _N_CHUNKS - _BASE_CHUNKS * _NW
_MAXC = _BASE_CHUNKS + 1
_UNROLL = 8


def _body(ah_hbm, bh_hbm, us_hbm, vs_hbm, ws_hbm, out_hbm,
          idx_u, idx_v, ws_v, out_v,
          rows_u0, rows_u1, rows_v0, rows_v1,
          sem_idx, sem_rows0, sem_rows1, sem_out):
    wid = lax.axis_index("s") * _NC + lax.axis_index("c")
    lane = lax.iota(jnp.int32, _L)
    rows_u = (rows_u0, rows_u1)
    rows_v = (rows_v0, rows_v1)
    sem_rows = (sem_rows0, sem_rows1)

    has_extra = wid < _EXTRA_W
    n_chunks = jnp.where(has_extra, _MAXC, _BASE_CHUNKS)
    start = _BASE_CHUNKS * wid + jnp.minimum(wid, _EXTRA_W)
    ebase = start * _K

    nmain = _BASE_CHUNKS * _K
    pltpu.make_async_copy(
        us_hbm.at[pl.ds(ebase, nmain)], idx_u.at[pl.ds(0, nmain)], sem_idx
    ).start()
    pltpu.make_async_copy(
        vs_hbm.at[pl.ds(ebase, nmain)], idx_v.at[pl.ds(0, nmain)], sem_idx
    ).start()
    pltpu.make_async_copy(
        ws_hbm.at[pl.ds(ebase, nmain)], ws_v.at[pl.ds(0, nmain)], sem_idx
    ).start()

    @pl.when(has_extra)
    def _():
        for hbm, vmem in ((us_hbm, idx_u), (vs_hbm, idx_v), (ws_hbm, ws_v)):
            pltpu.make_async_copy(
                hbm.at[pl.ds(ebase + nmain, _K)],
                vmem.at[pl.ds(nmain, _K)], sem_idx,
            ).start()

    def row_copies(m, p):
        off = m * _K
        return (
            pltpu.make_async_copy(
                ah_hbm.at[idx_u.at[pl.ds(off, _K)]], rows_u[p], sem_rows[p]),
            pltpu.make_async_copy(
                bh_hbm.at[idx_v.at[pl.ds(off, _K)]], rows_v[p], sem_rows[p]),
        )

    def compute(m, p):
        off = m * _K
        for g in range(_K // _L):
            row = lane + (g * _L)

            def dim_body(d0, acc):
                for k in range(_UNROLL):
                    col = (lane + (d0 * _UNROLL + k)) & (_D - 1)
                    gu = plsc.load_gather(rows_u[p], [row, col])
                    gv = plsc.load_gather(rows_v[p], [row, col])
                    cmb = gu + gv
                    acc = acc + cmb * cmb
                return acc

            acc = lax.fori_loop(0, _D // _UNROLL, dim_body,
                                jnp.zeros((_L,), jnp.float32))
            w = ws_v[pl.ds(off + g * _L, _L)]
            out_v[pl.ds(off + g * _L, _L)] = w / (1.0 + jnp.exp(10.0 - acc))

    drain = pltpu.make_async_copy(
        us_hbm.at[pl.ds(ebase, nmain)], idx_u.at[pl.ds(0, nmain)], sem_idx)
    for _i in range(3):
        drain.wait()

    @pl.when(has_extra)
    def _():
        d = pltpu.make_async_copy(
            us_hbm.at[pl.ds(ebase + nmain, _K)],
            idx_u.at[pl.ds(nmain, _K)], sem_idx)
        for _i in range(3):
            d.wait()

    for cp in row_copies(0, 0):
        cp.start()

    def process(m, p):
        @pl.when(m + 1 < n_chunks)
        def _():
            for cp in row_copies(m + 1, 1 - p):
                cp.start()

        @pl.when(m < n_chunks)
        def _():
            for cp in row_copies(m, p):
                cp.wait()
            compute(m, p)

    def pair_body(i2, carry):
        process(i2 * 2, 0)
        process(i2 * 2 + 1, 1)
        return carry

    lax.fori_loop(0, (_MAXC + 1) // 2, pair_body, 0)

    out_main = pltpu.make_async_copy(
        out_v.at[pl.ds(0, nmain)], out_hbm.at[pl.ds(ebase, nmain)], sem_out)
    out_main.start()

    @pl.when(has_extra)
    def _():
        pltpu.make_async_copy(
            out_v.at[pl.ds(nmain, _K)],
            out_hbm.at[pl.ds(ebase + nmain, _K)], sem_out,
        ).start()

    out_main.wait()

    @pl.when(has_extra)
    def _():
        pltpu.make_async_copy(
            out_v.at[pl.ds(nmain, _K)],
            out_hbm.at[pl.ds(ebase + nmain, _K)], sem_out,
        ).wait()


@jax.jit
def _scorer(ah, bh, us, vs, ws):
    mesh = plsc.VectorSubcoreMesh(
        core_axis_name="c", subcore_axis_name="s",
        num_cores=_NC, num_subcores=_NS)
    return pl.kernel(
        _body,
        out_type=jax.ShapeDtypeStruct((_N_EDGES,), jnp.float32),
        mesh=mesh,
        compiler_params=pltpu.CompilerParams(needs_layout_passes=False),
        scratch_types=[
            pltpu.VMEM((_MAXC * _K,), jnp.int32),
            pltpu.VMEM((_MAXC * _K,), jnp.int32),
            pltpu.VMEM((_MAXC * _K,), jnp.float32),
            pltpu.VMEM((_MAXC * _K,), jnp.float32),
            pltpu.VMEM((_K, _D), jnp.float32),
            pltpu.VMEM((_K, _D), jnp.float32),
            pltpu.VMEM((_K, _D), jnp.float32),
            pltpu.VMEM((_K, _D), jnp.float32),
            pltpu.SemaphoreType.DMA,
            pltpu.SemaphoreType.DMA,
            pltpu.SemaphoreType.DMA,
            pltpu.SemaphoreType.DMA,
        ],
    )(ah, bh, us, vs, ws)


def kernel(h, us, vs, ws, a, b):
    return _scorer(h * a, h * b, us, vs, ws)
